# baseline (speedup 1.0000x reference)
import functools

import jax
import jax.numpy as jnp
from jax import lax
from jax.experimental import pallas as pl
from jax.experimental.pallas import tpu as pltpu
from jax.experimental.pallas import tpu_sc as plsc

_KEEP_FRAC = 0.1

_HB = 32768
_LB = 65536
_CH = 8192
_SLOT = _HB


def _sc_threshold_body(k, nrows_s, ncols, x_hbm, out_hbm, hist, buf0, buf1,
                       mbuf, row16, rsbuf, shared, sem0, sem1, osem0, osem1):
    c = lax.axis_index("c")
    s = lax.axis_index("s")
    hi = s // 8
    part = s % 8
    sample = 2 * c + hi
    rows_p = nrows_s // 8
    ncq = ncols // 1024
    nch = (rows_p // 8) * ncq
    row0 = sample * nrows_s + part * rows_p
    slot = s * _SLOT
    merged = 16 * _SLOT + hi * _SLOT
    rs_base = 18 * _SLOT + hi * 256

    iota16 = lax.iota(jnp.int32, 16)
    ones16 = jnp.ones((16,), jnp.int32)
    zeros16 = jnp.zeros((16,), jnp.int32)
    kt = jnp.int32(k)

    def zero_hist(lo, nwords):
        @plsc.parallel_loop(0, nwords // 16, unroll=8)
        def _(j):
            hist[pl.ds(lo + j * 16, 16)] = zeros16

    def chunk_copy(ci, buf, sem):
        rg = ci // ncq
        cq = ci % ncq
        return pltpu.async_copy(
            x_hbm.at[pl.ds(row0 + rg * 8, 8), pl.ds(cq * 1024, 1024)],
            buf, sem)

    def chunk_wait(buf, sem):
        pltpu.make_async_copy(
            x_hbm.at[pl.ds(row0, 8), pl.ds(0, 1024)], buf, sem).wait()

    def out_copy(ci, buf, sem):
        rg = ci // ncq
        cq = ci % ncq
        return pltpu.async_copy(
            buf,
            out_hbm.at[pl.ds(row0 + rg * 8, 8), pl.ds(cq * 1024, 1024)],
            sem)

    def out_wait(buf, sem):
        pltpu.make_async_copy(
            buf, out_hbm.at[pl.ds(row0, 8), pl.ds(0, 1024)], sem).wait()

    def stream_pass(proc):
        chunk_copy(0, buf0, sem0)

        def body(t, _):
            chunk_copy(2 * t + 1, buf1, sem1)
            chunk_wait(buf0, sem0)
            proc(buf0)

            @pl.when(t < nch // 2 - 1)
            def _():
                chunk_copy(2 * t + 2, buf0, sem0)

            chunk_wait(buf1, sem1)
            proc(buf1)
            return 0

        lax.fori_loop(0, nch // 2, body, 0)

    def proc_a(buf):
        @plsc.parallel_loop(0, _CH // 16, unroll=8)
        def _(i):
            v = buf[i & 7, pl.ds((i >> 3) * 16, 16)]
            key = lax.bitcast_convert_type(v, jnp.int32) & jnp.int32(0x7FFFFFFF)
            plsc.addupdate_scatter(hist, [key >> 16], ones16)

    def make_proc_b(bstar_vec):
        def proc_b(buf):
            @plsc.parallel_loop(0, _CH // 16, unroll=8)
            def _(i):
                v = buf[i & 7, pl.ds((i >> 3) * 16, 16)]
                key = (lax.bitcast_convert_type(v, jnp.int32)
                       & jnp.int32(0x7FFFFFFF))
                m = (key >> 16) == bstar_vec
                plsc.addupdate_scatter(hist, [key & jnp.int32(0xFFFF)],
                                       ones16, mask=m)
        return proc_b

    def merge_pass(nbins):
        rng = _SLOT // 8
        r0 = part * rng
        for h in range(nbins // _SLOT):
            hb = h * _SLOT
            pltpu.sync_copy(hist.at[pl.ds(hb, _SLOT)],
                            shared.at[pl.ds(slot, _SLOT)])
            plsc.subcore_barrier()
            for o in range(7):
                other = hi * 8 + jnp.where(o < part, o, o + 1)

                pltpu.sync_copy(shared.at[pl.ds(other * _SLOT + r0, rng)],
                                mbuf.at[pl.ds(0, rng)])

                @plsc.parallel_loop(0, rng // 16, unroll=4)
                def _(j):
                    hist[pl.ds(hb + r0 + j * 16, 16)] = (
                        hist[pl.ds(hb + r0 + j * 16, 16)]
                        + mbuf[pl.ds(j * 16, 16)])
            pltpu.sync_copy(hist.at[pl.ds(hb + r0, rng)],
                            shared.at[pl.ds(merged + r0, rng)])
            rsvec = zeros16
            for rr in range(8):
                acc = plsc.parallel_loop(0, 32, unroll=4, carry=zeros16)(
                    lambda t, a, rr=rr:
                    a + hist[pl.ds(hb + r0 + rr * 512 + t * 16, 16)])
                rsvec = jnp.where(
                    iota16 == rr,
                    jnp.broadcast_to(jnp.sum(acc), (16,)).astype(jnp.int32),
                    rsvec)
            row16[...] = rsvec
            pltpu.sync_copy(
                row16, shared.at[pl.ds(rs_base + (h * 8) * 16 + part * 16,
                                       16)])
            plsc.subcore_barrier()
            pltpu.sync_copy(shared.at[pl.ds(merged, _SLOT)],
                            hist.at[pl.ds(hb, _SLOT)])

    def scan_hist(nbins, ktarget):
        nblk = (nbins // _SLOT) * 8
        pltpu.sync_copy(shared.at[pl.ds(rs_base, nblk * 16)],
                        rsbuf.at[pl.ds(0, nblk * 16)])

        def blk_body(q, carry):
            cum, r_star, c_above, done = carry
            b = nblk - 1 - q
            v = rsbuf[pl.ds(b * 16, 16)]
            rv = lax.rev(v, (0,))
            cs = plsc.cumsum(rv)
            hit = (cum + cs) >= ktarget
            pc = jnp.max(plsc.all_reduce_population_count(hit))
            ffs = jnp.max(plsc.all_reduce_ffs(hit))
            newly = jnp.logical_and(pc > 0, jnp.logical_not(done))
            prev = jnp.sum(jnp.where(iota16 == ffs, cs - rv, 0))
            r_star = jnp.where(newly, 8 * b + 15 - ffs, r_star)
            c_above = jnp.where(newly, cum + prev, c_above)
            done = jnp.logical_or(done, pc > 0)
            cum = cum + jnp.sum(v)
            return (cum, r_star, c_above, done)

        _, r_star, c_rows, _ = lax.fori_loop(
            0, nblk, blk_body,
            (jnp.int32(0), jnp.int32(0), jnp.int32(0), jnp.bool_(False)))

        def vec_body(q, carry):
            cum, w_star, c_above, done = carry
            t = 31 - q
            v = hist[pl.ds(r_star * 512 + t * 16, 16)]
            rv = lax.rev(v, (0,))
            cs = plsc.cumsum(rv)
            hit = (cum + cs) >= ktarget
            pc = jnp.max(plsc.all_reduce_population_count(hit))
            ffs = jnp.max(plsc.all_reduce_ffs(hit))
            newly = jnp.logical_and(pc > 0, jnp.logical_not(done))
            prev = jnp.sum(jnp.where(iota16 == ffs, cs - rv, 0))
            w_star = jnp.where(newly, t * 16 + 15 - ffs, w_star)
            c_above = jnp.where(newly, cum + prev, c_above)
            done = jnp.logical_or(done, pc > 0)
            cum = cum + jnp.sum(v)
            return (cum, w_star, c_above, done)

        _, w_star, c_above, _ = lax.fori_loop(
            0, 32, vec_body, (c_rows, jnp.int32(0), c_rows, jnp.bool_(False)))
        return r_star * 512 + w_star, c_above

    zero_hist(0, _HB)
    stream_pass(proc_a)
    zero_hist(_HB, _LB - _HB)
    merge_pass(_HB)
    bstar, c_above_a = scan_hist(_HB, kt)

    zero_hist(0, _HB)
    bstar_vec = jnp.broadcast_to(bstar, (16,)).astype(jnp.int32)
    stream_pass(make_proc_b(bstar_vec))
    merge_pass(_LB)
    vstar, _ = scan_hist(_LB, kt - c_above_a)
    thr = bstar * jnp.int32(65536) + vstar

    thrv = jnp.broadcast_to(thr, (16,)).astype(jnp.int32)

    def proc_c(buf):
        @plsc.parallel_loop(0, _CH // 16, unroll=8)
        def _(i):
            v = buf[i & 7, pl.ds((i >> 3) * 16, 16)]
            key = lax.bitcast_convert_type(v, jnp.int32) & jnp.int32(0x7FFFFFFF)
            buf[i & 7, pl.ds((i >> 3) * 16, 16)] = jnp.where(
                key >= thrv, v, jnp.float32(0.0))

    chunk_copy(0, buf0, sem0)
    chunk_copy(1, buf1, sem1)

    def cbody(t, _):
        chunk_wait(buf0, sem0)
        proc_c(buf0)
        out_copy(2 * t, buf0, osem0)
        chunk_wait(buf1, sem1)
        proc_c(buf1)
        out_copy(2 * t + 1, buf1, osem1)

        @pl.when(t < nch // 2 - 1)
        def _():
            out_wait(buf0, osem0)
            chunk_copy(2 * t + 2, buf0, sem0)
            out_wait(buf1, osem1)
            chunk_copy(2 * t + 3, buf1, sem1)

        return 0

    lax.fori_loop(0, nch // 2, cbody, 0)
    out_wait(buf0, osem0)
    out_wait(buf1, osem1)


def _sc_threshold(x2d, k, nrows_s, ncols):
    mesh = plsc.VectorSubcoreMesh(core_axis_name="c", subcore_axis_name="s")
    f = pl.kernel(
        functools.partial(_sc_threshold_body, k, nrows_s, ncols),
        out_type=jax.ShapeDtypeStruct(x2d.shape, x2d.dtype),
        mesh=mesh,
        compiler_params=pltpu.CompilerParams(use_tc_tiling_on_sc=True,
                                             needs_layout_passes=False),
        scratch_types=[
            pltpu.VMEM((_LB,), jnp.int32),
            pltpu.VMEM((8, 1024), jnp.float32),
            pltpu.VMEM((8, 1024), jnp.float32),
            pltpu.VMEM((_CH,), jnp.int32),
            pltpu.VMEM((16,), jnp.int32),
            pltpu.VMEM((256,), jnp.int32),
            pltpu.VMEM_SHARED((18 * _SLOT + 512,), jnp.int32),
            pltpu.SemaphoreType.DMA,
            pltpu.SemaphoreType.DMA,
            pltpu.SemaphoreType.DMA,
            pltpu.SemaphoreType.DMA,
        ],
    )
    return f(x2d)


def kernel(x):
    B, C, L = x.shape
    n = C * L
    k = max(1, int(round(_KEEP_FRAC * n)))
    return _sc_threshold(x.reshape(B * C, L), k, C, L).reshape(B, C, L)


# speedup vs baseline: 1.0624x; 1.0624x over previous
import functools

import jax
import jax.numpy as jnp
from jax import lax
from jax.experimental import pallas as pl
from jax.experimental.pallas import tpu as pltpu
from jax.experimental.pallas import tpu_sc as plsc

_KEEP_FRAC = 0.1

_HB = 32768
_LB = 65536
_CH = 8192
_SLOT = _HB


def _sc_threshold_body(k, nrows_s, ncols, x_hbm, out_hbm, hist, buf0, buf1,
                       mbuf, row16, rsbuf, shared, sem0, sem1):
    c = lax.axis_index("c")
    s = lax.axis_index("s")
    hi = s // 8
    part = s % 8
    sample = 2 * c + hi
    rows_p = nrows_s // 8
    ncq = ncols // 1024
    nch = (rows_p // 8) * ncq
    row0 = sample * nrows_s + part * rows_p
    slot = s * _SLOT
    merged = 16 * _SLOT + hi * _SLOT
    rs_base = 18 * _SLOT + hi * 256

    iota16 = lax.iota(jnp.int32, 16)
    ones16 = jnp.ones((16,), jnp.int32)
    zeros16 = jnp.zeros((16,), jnp.int32)
    kt = jnp.int32(k)

    def zero_hist(lo, nwords):
        @plsc.parallel_loop(0, nwords // 16, unroll=8)
        def _(j):
            hist[pl.ds(lo + j * 16, 16)] = zeros16

    def stream_pass(proc):
        def chunk_copy(ci, buf, sem):
            rg = ci // ncq
            cq = ci % ncq
            return pltpu.async_copy(
                x_hbm.at[pl.ds(row0 + rg * 8, 8), pl.ds(cq * 1024, 1024)],
                buf, sem)

        def chunk_wait(buf, sem):
            pltpu.make_async_copy(
                x_hbm.at[pl.ds(row0, 8), pl.ds(0, 1024)], buf, sem).wait()

        chunk_copy(0, buf0, sem0)

        def body(t, _):
            chunk_copy(2 * t + 1, buf1, sem1)
            chunk_wait(buf0, sem0)
            proc(buf0)

            @pl.when(t < nch // 2 - 1)
            def _():
                chunk_copy(2 * t + 2, buf0, sem0)

            chunk_wait(buf1, sem1)
            proc(buf1)
            return 0

        lax.fori_loop(0, nch // 2, body, 0)

    def proc_a(buf):
        @plsc.parallel_loop(0, _CH // 16, unroll=8)
        def _(i):
            v = buf[i & 7, pl.ds((i >> 3) * 16, 16)]
            key = lax.bitcast_convert_type(v, jnp.int32) & jnp.int32(0x7FFFFFFF)
            plsc.addupdate_scatter(hist, [key >> 16], ones16)

    def make_proc_b(bstar_vec):
        def proc_b(buf):
            @plsc.parallel_loop(0, _CH // 16, unroll=8)
            def _(i):
                v = buf[i & 7, pl.ds((i >> 3) * 16, 16)]
                key = (lax.bitcast_convert_type(v, jnp.int32)
                       & jnp.int32(0x7FFFFFFF))
                m = (key >> 16) == bstar_vec
                plsc.addupdate_scatter(hist, [key & jnp.int32(0xFFFF)],
                                       ones16, mask=m)
        return proc_b

    def merge_pass(nbins):
        rng = _SLOT // 8
        r0 = part * rng
        for h in range(nbins // _SLOT):
            hb = h * _SLOT
            pltpu.sync_copy(hist.at[pl.ds(hb, _SLOT)],
                            shared.at[pl.ds(slot, _SLOT)])
            plsc.subcore_barrier()
            for o in range(7):
                other = hi * 8 + jnp.where(o < part, o, o + 1)

                pltpu.sync_copy(shared.at[pl.ds(other * _SLOT + r0, rng)],
                                mbuf.at[pl.ds(0, rng)])

                @plsc.parallel_loop(0, rng // 16, unroll=4)
                def _(j):
                    hist[pl.ds(hb + r0 + j * 16, 16)] = (
                        hist[pl.ds(hb + r0 + j * 16, 16)]
                        + mbuf[pl.ds(j * 16, 16)])
            pltpu.sync_copy(hist.at[pl.ds(hb + r0, rng)],
                            shared.at[pl.ds(merged + r0, rng)])
            rsvec = zeros16
            for rr in range(8):
                acc = plsc.parallel_loop(0, 32, unroll=4, carry=zeros16)(
                    lambda t, a, rr=rr:
                    a + hist[pl.ds(hb + r0 + rr * 512 + t * 16, 16)])
                rsvec = jnp.where(
                    iota16 == rr,
                    jnp.broadcast_to(jnp.sum(acc), (16,)).astype(jnp.int32),
                    rsvec)
            row16[...] = rsvec
            pltpu.sync_copy(
                row16, shared.at[pl.ds(rs_base + (h * 8) * 16 + part * 16,
                                       16)])
            plsc.subcore_barrier()
            pltpu.sync_copy(shared.at[pl.ds(merged, _SLOT)],
                            hist.at[pl.ds(hb, _SLOT)])

    def scan_hist(nbins, ktarget):
        nblk = (nbins // _SLOT) * 8
        pltpu.sync_copy(shared.at[pl.ds(rs_base, nblk * 16)],
                        rsbuf.at[pl.ds(0, nblk * 16)])

        def blk_body(q, carry):
            cum, r_star, c_above, done = carry
            b = nblk - 1 - q
            v = rsbuf[pl.ds(b * 16, 16)]
            rv = lax.rev(v, (0,))
            cs = plsc.cumsum(rv)
            hit = (cum + cs) >= ktarget
            pc = jnp.max(plsc.all_reduce_population_count(hit))
            ffs = jnp.max(plsc.all_reduce_ffs(hit))
            newly = jnp.logical_and(pc > 0, jnp.logical_not(done))
            prev = jnp.sum(jnp.where(iota16 == ffs, cs - rv, 0))
            r_star = jnp.where(newly, 8 * b + 15 - ffs, r_star)
            c_above = jnp.where(newly, cum + prev, c_above)
            done = jnp.logical_or(done, pc > 0)
            cum = cum + jnp.sum(v)
            return (cum, r_star, c_above, done)

        _, r_star, c_rows, _ = lax.fori_loop(
            0, nblk, blk_body,
            (jnp.int32(0), jnp.int32(0), jnp.int32(0), jnp.bool_(False)))

        def vec_body(q, carry):
            cum, w_star, c_above, done = carry
            t = 31 - q
            v = hist[pl.ds(r_star * 512 + t * 16, 16)]
            rv = lax.rev(v, (0,))
            cs = plsc.cumsum(rv)
            hit = (cum + cs) >= ktarget
            pc = jnp.max(plsc.all_reduce_population_count(hit))
            ffs = jnp.max(plsc.all_reduce_ffs(hit))
            newly = jnp.logical_and(pc > 0, jnp.logical_not(done))
            prev = jnp.sum(jnp.where(iota16 == ffs, cs - rv, 0))
            w_star = jnp.where(newly, t * 16 + 15 - ffs, w_star)
            c_above = jnp.where(newly, cum + prev, c_above)
            done = jnp.logical_or(done, pc > 0)
            cum = cum + jnp.sum(v)
            return (cum, w_star, c_above, done)

        _, w_star, c_above, _ = lax.fori_loop(
            0, 32, vec_body, (c_rows, jnp.int32(0), c_rows, jnp.bool_(False)))
        return r_star * 512 + w_star, c_above

    zero_hist(0, _HB)
    stream_pass(proc_a)
    zero_hist(_HB, _LB - _HB)
    merge_pass(_HB)
    bstar, c_above_a = scan_hist(_HB, kt)

    zero_hist(0, _HB)
    bstar_vec = jnp.broadcast_to(bstar, (16,)).astype(jnp.int32)
    stream_pass(make_proc_b(bstar_vec))
    merge_pass(_LB)
    vstar, _ = scan_hist(_LB, kt - c_above_a)
    thr = bstar * jnp.int32(65536) + vstar

    @pl.when(part == 0)
    def _():
        row16[...] = jnp.broadcast_to(thr, (16,)).astype(jnp.int32)
        pltpu.sync_copy(row16, out_hbm.at[pl.ds(sample * 16, 16)])


def _sc_threshold(x2d, k, nrows_s, ncols):
    mesh = plsc.VectorSubcoreMesh(core_axis_name="c", subcore_axis_name="s")
    f = pl.kernel(
        functools.partial(_sc_threshold_body, k, nrows_s, ncols),
        out_type=jax.ShapeDtypeStruct((64,), jnp.int32),
        mesh=mesh,
        compiler_params=pltpu.CompilerParams(use_tc_tiling_on_sc=True,
                                             needs_layout_passes=False),
        scratch_types=[
            pltpu.VMEM((_LB,), jnp.int32),
            pltpu.VMEM((8, 1024), jnp.float32),
            pltpu.VMEM((8, 1024), jnp.float32),
            pltpu.VMEM((_CH,), jnp.int32),
            pltpu.VMEM((16,), jnp.int32),
            pltpu.VMEM((256,), jnp.int32),
            pltpu.VMEM_SHARED((18 * _SLOT + 512,), jnp.int32),
            pltpu.SemaphoreType.DMA,
            pltpu.SemaphoreType.DMA,
        ],
    )
    return f(x2d)


def _mask_body(thr_ref, x_ref, o_ref):
    b = pl.program_id(0)
    t = thr_ref[b * 16]
    x = x_ref[...]
    keys = lax.bitcast_convert_type(x, jnp.int32) & jnp.int32(0x7FFFFFFF)
    o_ref[...] = jnp.where(keys >= t, x, jnp.float32(0.0))


def kernel(x):
    B, C, L = x.shape
    n = C * L
    k = max(1, int(round(_KEEP_FRAC * n)))
    thr = _sc_threshold(x.reshape(B * C, L), k, C, L)
    lc = 1024
    return pl.pallas_call(
        _mask_body,
        grid=(B, L // lc),
        in_specs=[
            pl.BlockSpec(memory_space=pltpu.SMEM),
            pl.BlockSpec((1, C, lc), lambda b, j: (b, 0, j)),
        ],
        out_specs=pl.BlockSpec((1, C, lc), lambda b, j: (b, 0, j)),
        out_shape=jax.ShapeDtypeStruct(x.shape, x.dtype),
    )(thr, x)


# speedup vs baseline: 1.0881x; 1.0243x over previous
import functools

import jax
import jax.numpy as jnp
from jax import lax
from jax.experimental import pallas as pl
from jax.experimental.pallas import tpu as pltpu
from jax.experimental.pallas import tpu_sc as plsc

_KEEP_FRAC = 0.1

_HB = 32768
_LB = 65536
_CH = 8192
_SLOT = _HB


def _sc_threshold_body(k, nrows_s, ncols, x_hbm, out_hbm, hist, buf0, buf1,
                       mbuf, row16, rsbuf, shared, sem0, sem1):
    c = lax.axis_index("c")
    s = lax.axis_index("s")
    hi = s // 8
    part = s % 8
    sample = 2 * c + hi
    rows_p = nrows_s // 8
    ncq = ncols // 1024
    nch = (rows_p // 8) * ncq
    row0 = sample * nrows_s + part * rows_p
    slot = s * _SLOT
    merged = 16 * _SLOT + hi * _SLOT
    rs_base = 18 * _SLOT + hi * 256

    iota16 = lax.iota(jnp.int32, 16)
    ones16 = jnp.ones((16,), jnp.int32)
    zeros16 = jnp.zeros((16,), jnp.int32)
    kt = jnp.int32(k)

    def zero_hist(lo, nwords):
        @plsc.parallel_loop(0, nwords // 16, unroll=8)
        def _(j):
            hist[pl.ds(lo + j * 16, 16)] = zeros16

    def stream_pass(proc):
        def chunk_copy(ci, buf, sem):
            rg = ci // ncq
            cq = ci % ncq
            return pltpu.async_copy(
                x_hbm.at[pl.ds(row0 + rg * 8, 8), pl.ds(cq * 1024, 1024)],
                buf, sem)

        def chunk_wait(buf, sem):
            pltpu.make_async_copy(
                x_hbm.at[pl.ds(row0, 8), pl.ds(0, 1024)], buf, sem).wait()

        chunk_copy(0, buf0, sem0)

        def body(t, _):
            chunk_copy(2 * t + 1, buf1, sem1)
            chunk_wait(buf0, sem0)
            proc(buf0)

            @pl.when(t < nch // 2 - 1)
            def _():
                chunk_copy(2 * t + 2, buf0, sem0)

            chunk_wait(buf1, sem1)
            proc(buf1)
            return 0

        lax.fori_loop(0, nch // 2, body, 0)

    def proc_a(buf):
        @plsc.parallel_loop(0, _CH // 16, unroll=8)
        def _(i):
            v = buf[i & 7, pl.ds((i >> 3) * 16, 16)]
            key = lax.bitcast_convert_type(v, jnp.int32) & jnp.int32(0x7FFFFFFF)
            plsc.addupdate_scatter(hist, [key >> 16], ones16)

    def make_proc_b(bstar_vec):
        def proc_b(buf):
            @plsc.parallel_loop(0, _CH // 16, unroll=8)
            def _(i):
                v = buf[i & 7, pl.ds((i >> 3) * 16, 16)]
                key = (lax.bitcast_convert_type(v, jnp.int32)
                       & jnp.int32(0x7FFFFFFF))
                m = (key >> 16) == bstar_vec
                plsc.addupdate_scatter(hist, [key & jnp.int32(0xFFFF)],
                                       ones16, mask=m)
        return proc_b

    def merge_pass(nbins):
        rng = _SLOT // 8
        r0 = part * rng
        for h in range(nbins // _SLOT):
            hb = h * _SLOT
            pltpu.sync_copy(hist.at[pl.ds(hb, _SLOT)],
                            shared.at[pl.ds(slot, _SLOT)])
            plsc.subcore_barrier()
            for o in range(7):
                other = hi * 8 + jnp.where(o < part, o, o + 1)

                pltpu.sync_copy(shared.at[pl.ds(other * _SLOT + r0, rng)],
                                mbuf.at[pl.ds(0, rng)])

                @plsc.parallel_loop(0, rng // 16, unroll=4)
                def _(j):
                    hist[pl.ds(hb + r0 + j * 16, 16)] = (
                        hist[pl.ds(hb + r0 + j * 16, 16)]
                        + mbuf[pl.ds(j * 16, 16)])
            pltpu.sync_copy(hist.at[pl.ds(hb + r0, rng)],
                            shared.at[pl.ds(merged + r0, rng)])
            rsvec = zeros16
            for rr in range(8):
                acc = plsc.parallel_loop(0, 32, unroll=4, carry=zeros16)(
                    lambda t, a, rr=rr:
                    a + hist[pl.ds(hb + r0 + rr * 512 + t * 16, 16)])
                rsvec = jnp.where(
                    iota16 == rr,
                    jnp.broadcast_to(jnp.sum(acc), (16,)).astype(jnp.int32),
                    rsvec)
            row16[...] = rsvec
            pltpu.sync_copy(
                row16, shared.at[pl.ds(rs_base + (h * 8) * 16 + part * 16,
                                       16)])
            plsc.subcore_barrier()
            pltpu.sync_copy(shared.at[pl.ds(merged, _SLOT)],
                            hist.at[pl.ds(hb, _SLOT)])

    def scan_hist(nbins, ktarget):
        nblk = (nbins // _SLOT) * 8
        pltpu.sync_copy(shared.at[pl.ds(rs_base, nblk * 16)],
                        rsbuf.at[pl.ds(0, nblk * 16)])

        def blk_body(q, carry):
            cum, r_star, c_above, done = carry
            b = nblk - 1 - q
            v = rsbuf[pl.ds(b * 16, 16)]
            rv = lax.rev(v, (0,))
            cs = plsc.cumsum(rv)
            hit = (cum + cs) >= ktarget
            pc = jnp.max(plsc.all_reduce_population_count(hit))
            ffs = jnp.max(plsc.all_reduce_ffs(hit))
            newly = jnp.logical_and(pc > 0, jnp.logical_not(done))
            prev = jnp.sum(jnp.where(iota16 == ffs, cs - rv, 0))
            r_star = jnp.where(newly, 8 * b + 15 - ffs, r_star)
            c_above = jnp.where(newly, cum + prev, c_above)
            done = jnp.logical_or(done, pc > 0)
            cum = cum + jnp.sum(v)
            return (cum, r_star, c_above, done)

        _, r_star, c_rows, _ = lax.fori_loop(
            0, nblk, blk_body,
            (jnp.int32(0), jnp.int32(0), jnp.int32(0), jnp.bool_(False)))

        def vec_body(q, carry):
            cum, w_star, c_above, done = carry
            t = 31 - q
            v = hist[pl.ds(r_star * 512 + t * 16, 16)]
            rv = lax.rev(v, (0,))
            cs = plsc.cumsum(rv)
            hit = (cum + cs) >= ktarget
            pc = jnp.max(plsc.all_reduce_population_count(hit))
            ffs = jnp.max(plsc.all_reduce_ffs(hit))
            newly = jnp.logical_and(pc > 0, jnp.logical_not(done))
            prev = jnp.sum(jnp.where(iota16 == ffs, cs - rv, 0))
            w_star = jnp.where(newly, t * 16 + 15 - ffs, w_star)
            c_above = jnp.where(newly, cum + prev, c_above)
            done = jnp.logical_or(done, pc > 0)
            cum = cum + jnp.sum(v)
            return (cum, w_star, c_above, done)

        _, w_star, c_above, _ = lax.fori_loop(
            0, 32, vec_body, (c_rows, jnp.int32(0), c_rows, jnp.bool_(False)))
        return r_star * 512 + w_star, c_above

    zero_hist(0, _HB)
    stream_pass(proc_a)
    zero_hist(_HB, _LB - _HB)
    merge_pass(_HB)
    bstar, c_above_a = scan_hist(_HB, kt)

    zero_hist(0, _HB)
    bstar_vec = jnp.broadcast_to(bstar, (16,)).astype(jnp.int32)
    stream_pass(make_proc_b(bstar_vec))
    merge_pass(_LB)
    vstar, _ = scan_hist(_LB, kt - c_above_a)
    thr = bstar * jnp.int32(65536) + vstar

    @pl.when(part == 0)
    def _():
        row16[...] = jnp.broadcast_to(thr, (16,)).astype(jnp.int32)
        pltpu.sync_copy(row16, out_hbm.at[pl.ds(sample * 16, 16)])


def _sc_threshold(x2d, k, nrows_s, ncols):
    mesh = plsc.VectorSubcoreMesh(core_axis_name="c", subcore_axis_name="s")
    f = pl.kernel(
        functools.partial(_sc_threshold_body, k, nrows_s, ncols),
        out_type=jax.ShapeDtypeStruct((64,), jnp.int32),
        mesh=mesh,
        compiler_params=pltpu.CompilerParams(use_tc_tiling_on_sc=True,
                                             needs_layout_passes=False),
        scratch_types=[
            pltpu.VMEM((_LB,), jnp.int32),
            pltpu.VMEM((8, 1024), jnp.float32),
            pltpu.VMEM((8, 1024), jnp.float32),
            pltpu.VMEM((_CH,), jnp.int32),
            pltpu.VMEM((16,), jnp.int32),
            pltpu.VMEM((256,), jnp.int32),
            pltpu.VMEM_SHARED((18 * _SLOT + 512,), jnp.int32),
            pltpu.SemaphoreType.DMA,
            pltpu.SemaphoreType.DMA,
        ],
    )
    return f(x2d)


def _mask_body(thr_ref, x_ref, o_ref):
    b = pl.program_id(0)
    t = thr_ref[b * 16]
    x = x_ref[...]
    keys = lax.bitcast_convert_type(x, jnp.int32) & jnp.int32(0x7FFFFFFF)
    o_ref[...] = jnp.where(keys >= t, x, jnp.float32(0.0))


def kernel(x):
    B, C, L = x.shape
    n = C * L
    k = max(1, int(round(_KEEP_FRAC * n)))
    thr = _sc_threshold(x.reshape(B * C, L), k, C, L)
    lc = 2048
    return pl.pallas_call(
        _mask_body,
        grid=(B, L // lc),
        in_specs=[
            pl.BlockSpec(memory_space=pltpu.SMEM),
            pl.BlockSpec((1, C, lc), lambda b, j: (b, 0, j)),
        ],
        out_specs=pl.BlockSpec((1, C, lc), lambda b, j: (b, 0, j)),
        out_shape=jax.ShapeDtypeStruct(x.shape, x.dtype),
    )(thr, x)


# speedup vs baseline: 1.1040x; 1.0146x over previous
import functools

import jax
import jax.numpy as jnp
from jax import lax
from jax.experimental import pallas as pl
from jax.experimental.pallas import tpu as pltpu
from jax.experimental.pallas import tpu_sc as plsc

_KEEP_FRAC = 0.1

_HB = 32768
_LB = 65536
_CH = 8192
_SLOT = _HB


def _sc_threshold_body(k, nrows_s, ncols, x_hbm, out_hbm, hist, buf0, buf1,
                       mbuf, row16, rsbuf, shared, sem0, sem1):
    c = lax.axis_index("c")
    s = lax.axis_index("s")
    hi = s // 8
    part = s % 8
    sample = 2 * c + hi
    rows_p = nrows_s // 8
    ncq = ncols // 1024
    nch = (rows_p // 8) * ncq
    row0 = sample * nrows_s + part * rows_p
    slot = s * _SLOT
    merged = 16 * _SLOT + hi * _SLOT
    rs_base = 18 * _SLOT + hi * 256

    iota16 = lax.iota(jnp.int32, 16)
    ones16 = jnp.ones((16,), jnp.int32)
    zeros16 = jnp.zeros((16,), jnp.int32)
    kt = jnp.int32(k)

    def zero_hist(lo, nwords):
        @plsc.parallel_loop(0, nwords // 16, unroll=8)
        def _(j):
            hist[pl.ds(lo + j * 16, 16)] = zeros16

    def stream_pass(proc):
        def chunk_copy(ci, buf, sem):
            rg = ci // ncq
            cq = ci % ncq
            return pltpu.async_copy(
                x_hbm.at[pl.ds(row0 + rg * 8, 8), pl.ds(cq * 1024, 1024)],
                buf, sem)

        def chunk_wait(buf, sem):
            pltpu.make_async_copy(
                x_hbm.at[pl.ds(row0, 8), pl.ds(0, 1024)], buf, sem).wait()

        chunk_copy(0, buf0, sem0)

        def body(t, _):
            chunk_copy(2 * t + 1, buf1, sem1)
            chunk_wait(buf0, sem0)
            proc(buf0)

            @pl.when(t < nch // 2 - 1)
            def _():
                chunk_copy(2 * t + 2, buf0, sem0)

            chunk_wait(buf1, sem1)
            proc(buf1)
            return 0

        lax.fori_loop(0, nch // 2, body, 0)

    def proc_a(buf):
        @plsc.parallel_loop(0, _CH // 16, unroll=8)
        def _(i):
            v = buf[i & 7, pl.ds((i >> 3) * 16, 16)]
            key = lax.bitcast_convert_type(v, jnp.int32) & jnp.int32(0x7FFFFFFF)
            plsc.addupdate_scatter(hist, [key >> 16], ones16)

    def make_proc_b(bstar_vec):
        def proc_b(buf):
            @plsc.parallel_loop(0, _CH // 16, unroll=8)
            def _(i):
                v = buf[i & 7, pl.ds((i >> 3) * 16, 16)]
                key = (lax.bitcast_convert_type(v, jnp.int32)
                       & jnp.int32(0x7FFFFFFF))
                m = (key >> 16) == bstar_vec
                plsc.addupdate_scatter(hist, [key & jnp.int32(0xFFFF)],
                                       ones16, mask=m)
        return proc_b

    def merge_pass(nbins):
        rng = _SLOT // 8
        r0 = part * rng
        for h in range(nbins // _SLOT):
            hb = h * _SLOT
            pltpu.sync_copy(hist.at[pl.ds(hb, _SLOT)],
                            shared.at[pl.ds(slot, _SLOT)])
            plsc.subcore_barrier()
            for o in range(7):
                other = hi * 8 + jnp.where(o < part, o, o + 1)

                pltpu.sync_copy(shared.at[pl.ds(other * _SLOT + r0, rng)],
                                mbuf.at[pl.ds(0, rng)])

                @plsc.parallel_loop(0, rng // 16, unroll=4)
                def _(j):
                    hist[pl.ds(hb + r0 + j * 16, 16)] = (
                        hist[pl.ds(hb + r0 + j * 16, 16)]
                        + mbuf[pl.ds(j * 16, 16)])
            pltpu.sync_copy(hist.at[pl.ds(hb + r0, rng)],
                            shared.at[pl.ds(merged + r0, rng)])
            rsvec = zeros16
            for rr in range(8):
                acc = plsc.parallel_loop(0, 32, unroll=4, carry=zeros16)(
                    lambda t, a, rr=rr:
                    a + hist[pl.ds(hb + r0 + rr * 512 + t * 16, 16)])
                rsvec = jnp.where(
                    iota16 == rr,
                    jnp.broadcast_to(jnp.sum(acc), (16,)).astype(jnp.int32),
                    rsvec)
            row16[...] = rsvec
            pltpu.sync_copy(
                row16, shared.at[pl.ds(rs_base + (h * 8) * 16 + part * 16,
                                       16)])
            plsc.subcore_barrier()
            pltpu.sync_copy(shared.at[pl.ds(merged, _SLOT)],
                            hist.at[pl.ds(hb, _SLOT)])

    def scan_hist(nbins, ktarget):
        nblk = (nbins // _SLOT) * 8
        pltpu.sync_copy(shared.at[pl.ds(rs_base, nblk * 16)],
                        rsbuf.at[pl.ds(0, nblk * 16)])

        def blk_body(q, carry):
            cum, r_star, c_above, done = carry
            b = nblk - 1 - q
            v = rsbuf[pl.ds(b * 16, 16)]
            rv = lax.rev(v, (0,))
            cs = plsc.cumsum(rv)
            hit = (cum + cs) >= ktarget
            pc = jnp.max(plsc.all_reduce_population_count(hit))
            ffs = jnp.max(plsc.all_reduce_ffs(hit))
            newly = jnp.logical_and(pc > 0, jnp.logical_not(done))
            prev = jnp.sum(jnp.where(iota16 == ffs, cs - rv, 0))
            r_star = jnp.where(newly, 8 * b + 15 - ffs, r_star)
            c_above = jnp.where(newly, cum + prev, c_above)
            done = jnp.logical_or(done, pc > 0)
            cum = cum + jnp.sum(v)
            return (cum, r_star, c_above, done)

        _, r_star, c_rows, _ = lax.fori_loop(
            0, nblk, blk_body,
            (jnp.int32(0), jnp.int32(0), jnp.int32(0), jnp.bool_(False)))

        def vec_body(q, carry):
            cum, w_star, c_above, done = carry
            t = 31 - q
            v = hist[pl.ds(r_star * 512 + t * 16, 16)]
            rv = lax.rev(v, (0,))
            cs = plsc.cumsum(rv)
            hit = (cum + cs) >= ktarget
            pc = jnp.max(plsc.all_reduce_population_count(hit))
            ffs = jnp.max(plsc.all_reduce_ffs(hit))
            newly = jnp.logical_and(pc > 0, jnp.logical_not(done))
            prev = jnp.sum(jnp.where(iota16 == ffs, cs - rv, 0))
            w_star = jnp.where(newly, t * 16 + 15 - ffs, w_star)
            c_above = jnp.where(newly, cum + prev, c_above)
            done = jnp.logical_or(done, pc > 0)
            cum = cum + jnp.sum(v)
            return (cum, w_star, c_above, done)

        _, w_star, c_above, _ = lax.fori_loop(
            0, 32, vec_body, (c_rows, jnp.int32(0), c_rows, jnp.bool_(False)))
        return r_star * 512 + w_star, c_above

    zero_hist(0, _HB)
    stream_pass(proc_a)
    zero_hist(_HB, _LB - _HB)
    merge_pass(_HB)
    bstar, c_above_a = scan_hist(_HB, kt)

    zero_hist(0, _HB)
    bstar_vec = jnp.broadcast_to(bstar, (16,)).astype(jnp.int32)
    stream_pass(make_proc_b(bstar_vec))
    merge_pass(_LB)
    vstar, _ = scan_hist(_LB, kt - c_above_a)
    thr = bstar * jnp.int32(65536) + vstar

    @pl.when(part == 0)
    def _():
        row16[...] = jnp.broadcast_to(thr, (16,)).astype(jnp.int32)
        pltpu.sync_copy(row16, out_hbm.at[pl.ds(sample * 16, 16)])


def _sc_threshold(x2d, k, nrows_s, ncols):
    mesh = plsc.VectorSubcoreMesh(core_axis_name="c", subcore_axis_name="s")
    f = pl.kernel(
        functools.partial(_sc_threshold_body, k, nrows_s, ncols),
        out_type=jax.ShapeDtypeStruct((64,), jnp.int32),
        mesh=mesh,
        compiler_params=pltpu.CompilerParams(use_tc_tiling_on_sc=True,
                                             needs_layout_passes=False),
        scratch_types=[
            pltpu.VMEM((_LB,), jnp.int32),
            pltpu.VMEM((8, 1024), jnp.float32),
            pltpu.VMEM((8, 1024), jnp.float32),
            pltpu.VMEM((_CH,), jnp.int32),
            pltpu.VMEM((16,), jnp.int32),
            pltpu.VMEM((256,), jnp.int32),
            pltpu.VMEM_SHARED((18 * _SLOT + 512,), jnp.int32),
            pltpu.SemaphoreType.DMA,
            pltpu.SemaphoreType.DMA,
        ],
    )
    return f(x2d)


def _mask_body(thr_ref, x_ref, o_ref):
    b = pl.program_id(0)
    t = thr_ref[b * 16]
    x = x_ref[...]
    keys = lax.bitcast_convert_type(x, jnp.int32) & jnp.int32(0x7FFFFFFF)
    o_ref[...] = jnp.where(keys >= t, x, jnp.float32(0.0))


def kernel(x):
    B, C, L = x.shape
    n = C * L
    k = max(1, int(round(_KEEP_FRAC * n)))
    thr = _sc_threshold(x.reshape(B * C, L), k, C, L)
    lc = 4096
    return pl.pallas_call(
        _mask_body,
        grid=(B, L // lc),
        in_specs=[
            pl.BlockSpec(memory_space=pltpu.SMEM),
            pl.BlockSpec((1, C, lc), lambda b, j: (b, 0, j)),
        ],
        out_specs=pl.BlockSpec((1, C, lc), lambda b, j: (b, 0, j)),
        out_shape=jax.ShapeDtypeStruct(x.shape, x.dtype),
    )(thr, x)
